# Initial kernel scaffold; baseline (speedup 1.0000x reference)
import functools

import jax
import jax.numpy as jnp
from jax import lax
from jax.experimental import pallas as pl
from jax.experimental.pallas import tpu as pltpu
from jax.experimental.pallas import tpu_sc as plsc

B = 100
A = 100
N = B * A
D = 128
E = 320000
K = 8192

NC = 2
NS = 16
NW = NC * NS
EC = 128
NCHUNK = E // EC
CHUNKS_PER_TILE = -(-NCHUNK // NW)
NP = 10240
ROWS_PER_TILE = NP // NS
KC = K // (NW * EC)

_mesh = plsc.VectorSubcoreMesh(core_axis_name="c", subcore_axis_name="s")


def _edge_body(with_extras, *refs):
    if with_extras:
        (x_hbm, src_hbm, dst_hbm, x0_hbm, flat0_hbm, zD_hbm, o16_hbm, z16_hbm,
         agg_out, deg_out, d0_out,
         agg_sh, rows, zbuf, sidx, didx, sem, deg_sh, ones, zdeg) = refs
    else:
        (x_hbm, src_hbm, dst_hbm, zD_hbm,
         agg_out,
         agg_sh, rows, zbuf, sidx, didx, sem) = refs

    c = lax.axis_index("c")
    s = lax.axis_index("s")
    wid = s * NC + c

    pltpu.sync_copy(zD_hbm, zbuf)
    if with_extras:
        pltpu.sync_copy(o16_hbm, ones)
        pltpu.sync_copy(z16_hbm, zdeg)
    for k in range(ROWS_PER_TILE // EC):
        r = s * ROWS_PER_TILE + k * EC
        pltpu.sync_copy(zbuf, agg_sh.at[pl.ds(r, EC)])
        if with_extras:
            pltpu.sync_copy(zdeg, deg_sh.at[pl.ds(r, EC)])

    if with_extras:
        for k in range(KC):
            off = (wid * KC + k) * EC
            pltpu.sync_copy(flat0_hbm.at[pl.ds(off, EC)], sidx)
            pltpu.async_copy(x0_hbm.at[sidx], rows, sem).wait()
            pltpu.sync_copy(rows, d0_out.at[pl.ds(off, EC)])

    plsc.subcore_barrier()

    def chunk_body(j, carry):
        chunk = wid + NW * j

        @pl.when(chunk < NCHUNK)
        def _():
            base = chunk * EC
            pltpu.sync_copy(src_hbm.at[pl.ds(base, EC)], sidx)
            pltpu.sync_copy(dst_hbm.at[pl.ds(base, EC)], didx.at[0])
            pltpu.async_copy(x_hbm.at[sidx], rows, sem).wait()
            pltpu.sync_copy(rows, agg_sh.at[didx.at[0]], add=True)
            if with_extras:
                pltpu.sync_copy(ones, deg_sh.at[didx.at[0]], add=True)

        return carry

    lax.fori_loop(0, CHUNKS_PER_TILE, chunk_body, 0)

    plsc.subcore_barrier()

    for k in range(ROWS_PER_TILE // EC):
        r = s * ROWS_PER_TILE + k * EC
        pltpu.sync_copy(agg_sh.at[pl.ds(r, EC)], rows)
        pltpu.sync_copy(rows, agg_out.at[c, pl.ds(r, EC)])
        if with_extras:
            pltpu.sync_copy(deg_sh.at[pl.ds(r, EC)], ones)
            pltpu.sync_copy(ones, deg_out.at[c, pl.ds(r, EC)])


def _make_edge_kernel(with_extras):
    out_type = [jax.ShapeDtypeStruct((NC, NP, D), jnp.float32)]
    scratch = [
        pltpu.MemorySpace.VMEM_SHARED((NP, D), jnp.float32),
        pltpu.VMEM((EC, D), jnp.float32),
        pltpu.VMEM((EC, D), jnp.float32),
        pltpu.VMEM((EC,), jnp.int32),
        pltpu.VMEM((1, EC), jnp.int32),
        pltpu.SemaphoreType.DMA,
    ]
    if with_extras:
        out_type += [jax.ShapeDtypeStruct((NC, NP, 16), jnp.float32),
                     jax.ShapeDtypeStruct((K, D), jnp.float32)]
        scratch += [
            pltpu.MemorySpace.VMEM_SHARED((NP, 16), jnp.float32),
            pltpu.VMEM((EC, 16), jnp.float32),
            pltpu.VMEM((EC, 16), jnp.float32),
        ]
    return pl.kernel(
        functools.partial(_edge_body, with_extras),
        out_type=out_type,
        mesh=_mesh,
        scratch_types=scratch,
    )


def _gather_body(y_hbm, flat_hbm, out, sidx, rows, sem):
    c = lax.axis_index("c")
    s = lax.axis_index("s")
    wid = s * NC + c
    for k in range(KC):
        off = (wid * KC + k) * EC
        pltpu.sync_copy(flat_hbm.at[pl.ds(off, EC)], sidx)
        pltpu.async_copy(y_hbm.at[sidx], rows, sem).wait()
        pltpu.sync_copy(rows, out.at[pl.ds(off, EC)])


_gather_kernel = pl.kernel(
    _gather_body,
    out_type=[jax.ShapeDtypeStruct((K, D), jnp.float32)],
    mesh=_mesh,
    scratch_types=[
        pltpu.VMEM((EC,), jnp.int32),
        pltpu.VMEM((EC, D), jnp.float32),
        pltpu.SemaphoreType.DMA,
    ],
)

_ROWS_TC = 500


def _layer_tc_body(x_ref, aggp_ref, degp_ref, w_ref, b_ref, o_ref):
    agg = aggp_ref[0] + aggp_ref[1]
    deg = degp_ref[0] + degp_ref[1]
    rdeg = 1.0 / jnp.maximum(deg[:, 0:1], 1.0)
    h = jnp.dot(x_ref[...] + agg * rdeg, w_ref[...],
                preferred_element_type=jnp.float32) + b_ref[...]
    o_ref[...] = jnp.maximum(h, 0.0)


def _layer_tc(x, aggp, degp, w, b):
    grid = (N // _ROWS_TC,)
    return pl.pallas_call(
        _layer_tc_body,
        grid=grid,
        in_specs=[
            pl.BlockSpec((_ROWS_TC, D), lambda i: (i, 0)),
            pl.BlockSpec((NC, _ROWS_TC, D), lambda i: (0, i, 0)),
            pl.BlockSpec((NC, _ROWS_TC, 16), lambda i: (0, i, 0)),
            pl.BlockSpec((D, D), lambda i: (0, 0)),
            pl.BlockSpec((1, D), lambda i: (0, 0)),
        ],
        out_specs=pl.BlockSpec((_ROWS_TC, D), lambda i: (i, 0)),
        out_shape=jax.ShapeDtypeStruct((N, D), jnp.float32),
    )(x, aggp, degp, w, b)


_ROWS_FIN = 512


def _final_tc_body(d0_ref, d1_ref, fw_ref, fb_ref, lw_ref, lb_ref, o_ref):
    o_ref[0] = jnp.dot(d0_ref[...], fw_ref[...],
                       preferred_element_type=jnp.float32) + fb_ref[...]
    o_ref[1] = jnp.dot(d1_ref[...], lw_ref[...],
                       preferred_element_type=jnp.float32) + lb_ref[...]


def _final_tc(d0rows, d1rows, fw, fb, lw, lb):
    grid = (K // _ROWS_FIN,)
    return pl.pallas_call(
        _final_tc_body,
        grid=grid,
        in_specs=[
            pl.BlockSpec((_ROWS_FIN, D), lambda i: (i, 0)),
            pl.BlockSpec((_ROWS_FIN, D), lambda i: (i, 0)),
            pl.BlockSpec((D, D), lambda i: (0, 0)),
            pl.BlockSpec((1, D), lambda i: (0, 0)),
            pl.BlockSpec((D, D), lambda i: (0, 0)),
            pl.BlockSpec((1, D), lambda i: (0, 0)),
        ],
        out_specs=pl.BlockSpec((2, _ROWS_FIN, D), lambda i: (0, i, 0)),
        out_shape=jax.ShapeDtypeStruct((2, K, D), jnp.float32),
    )(d0rows, d1rows, fw, fb, lw, lb)


_edge_extras = _make_edge_kernel(True)
_edge_plain = _make_edge_kernel(False)


def kernel(x0, x1, edge_index1, adj1_0, adj1_1, gnn1_W, gnn1_b, gnn2_W,
           gnn2_b, lin_W, lin_b, fcin_W, fcin_b):
    src = edge_index1[0]
    dst = edge_index1[1]
    flat0 = adj1_0[:, 0] * A + adj1_0[:, 1]
    flat1 = adj1_1[:, 0] * A + adj1_1[:, 1]
    zD = jnp.zeros((EC, D), jnp.float32)
    o16 = jnp.ones((EC, 16), jnp.float32)
    z16 = jnp.zeros((EC, 16), jnp.float32)

    agg1, deg, d0rows = _edge_extras(x1, src, dst, x0, flat0, zD, o16, z16)
    y1 = _layer_tc(x1, agg1, deg, gnn1_W, gnn1_b.reshape(1, D))
    (agg2,) = _edge_plain(y1, src, dst, zD)
    y2 = _layer_tc(y1, agg2, deg, gnn2_W, gnn2_b.reshape(1, D))
    (d1rows,) = _gather_kernel(y2, flat1)
    return _final_tc(d0rows, d1rows, fcin_W, fcin_b.reshape(1, D),
                     lin_W, lin_b.reshape(1, D))


# speedup vs baseline: 5.5883x; 5.5883x over previous
import functools

import jax
import jax.numpy as jnp
from jax import lax
from jax.experimental import pallas as pl
from jax.experimental.pallas import tpu as pltpu
from jax.experimental.pallas import tpu_sc as plsc

B = 100
A = 100
N = B * A
D = 128
E = 320000
K = 8192

NC = 2
NS = 16
NW = NC * NS
EC = 128
NCHUNK = E // EC
CHUNKS_PER_TILE = -(-NCHUNK // NW)
NP = 10240
ROWS_PER_TILE = NP // NS
KC = K // (NW * EC)

_mesh = plsc.VectorSubcoreMesh(core_axis_name="c", subcore_axis_name="s")


def _zero_stripe(zD_hbm, rows, acc_sh, s):
    pltpu.sync_copy(zD_hbm, rows)
    for k in range(ROWS_PER_TILE // EC):
        pltpu.sync_copy(rows, acc_sh.at[pl.ds(s * ROWS_PER_TILE + k * EC, EC)])


def _copy_stripe_out(acc_sh, rows, out_hbm, c, s):
    for k in range(ROWS_PER_TILE // EC):
        r = s * ROWS_PER_TILE + k * EC
        pltpu.sync_copy(acc_sh.at[pl.ds(r, EC)], rows)
        pltpu.sync_copy(rows, out_hbm.at[c, pl.ds(r, EC)])


def _scatter_pass(dst_hbm, didx, agg_sh, wid, chunk_fn):
    def chunk_body(j, carry):
        chunk = wid + NW * j

        @pl.when(chunk < NCHUNK)
        def _():
            base = chunk * EC
            pltpu.sync_copy(dst_hbm.at[pl.ds(base, EC)], didx)
            src_rows = chunk_fn(base)
            pltpu.sync_copy(src_rows, agg_sh.at[didx], add=True)

        return carry

    lax.fori_loop(0, CHUNKS_PER_TILE, chunk_body, 0)


def _edge_body(with_extras, *refs):
    if with_extras:
        (x_hbm, src_hbm, dst_hbm, x0_hbm, flat0_hbm, zD_hbm, oD_hbm,
         agg_out, deg_out, d0_out,
         agg_sh, rows, sidx, didx, sem) = refs
    else:
        (x_hbm, src_hbm, dst_hbm, zD_hbm,
         agg_out,
         agg_sh, rows, sidx, didx, sem) = refs

    c = lax.axis_index("c")
    s = lax.axis_index("s")
    wid = s * NC + c

    _zero_stripe(zD_hbm, rows, agg_sh, s)

    if with_extras:
        pltpu.sync_copy(oD_hbm, rows)
        plsc.subcore_barrier()
        _scatter_pass(dst_hbm, didx, agg_sh, wid, lambda base: rows)
        plsc.subcore_barrier()
        _copy_stripe_out(agg_sh, rows, deg_out, c, s)
        _zero_stripe(zD_hbm, rows, agg_sh, s)

        for k in range(KC):
            off = (wid * KC + k) * EC
            pltpu.sync_copy(flat0_hbm.at[pl.ds(off, EC)], sidx)
            pltpu.async_copy(x0_hbm.at[sidx], rows, sem).wait()
            pltpu.sync_copy(rows, d0_out.at[pl.ds(off, EC)])

    plsc.subcore_barrier()

    def gather_rows(base):
        pltpu.sync_copy(src_hbm.at[pl.ds(base, EC)], sidx)
        pltpu.async_copy(x_hbm.at[sidx], rows, sem).wait()
        return rows

    _scatter_pass(dst_hbm, didx, agg_sh, wid, gather_rows)

    plsc.subcore_barrier()

    _copy_stripe_out(agg_sh, rows, agg_out, c, s)


def _make_edge_kernel(with_extras):
    out_type = [jax.ShapeDtypeStruct((NC, NP, D), jnp.float32)]
    scratch = [
        pltpu.MemorySpace.VMEM_SHARED((NP, D), jnp.float32),
        pltpu.VMEM((EC, D), jnp.float32),
        pltpu.VMEM((EC,), jnp.int32),
        pltpu.VMEM((EC,), jnp.int32),
        pltpu.SemaphoreType.DMA,
    ]
    if with_extras:
        out_type += [jax.ShapeDtypeStruct((NC, NP, D), jnp.float32),
                     jax.ShapeDtypeStruct((K, D), jnp.float32)]
    return pl.kernel(
        functools.partial(_edge_body, with_extras),
        out_type=out_type,
        mesh=_mesh,
        scratch_types=scratch,
    )


def _gather_body(y_hbm, flat_hbm, out, sidx, rows, sem):
    c = lax.axis_index("c")
    s = lax.axis_index("s")
    wid = s * NC + c
    for k in range(KC):
        off = (wid * KC + k) * EC
        pltpu.sync_copy(flat_hbm.at[pl.ds(off, EC)], sidx)
        pltpu.async_copy(y_hbm.at[sidx], rows, sem).wait()
        pltpu.sync_copy(rows, out.at[pl.ds(off, EC)])


_gather_kernel = pl.kernel(
    _gather_body,
    out_type=[jax.ShapeDtypeStruct((K, D), jnp.float32)],
    mesh=_mesh,
    scratch_types=[
        pltpu.VMEM((EC,), jnp.int32),
        pltpu.VMEM((EC, D), jnp.float32),
        pltpu.SemaphoreType.DMA,
    ],
)

_ROWS_TC = 400


def _layer_tc_body(x_ref, aggp_ref, degp_ref, w_ref, b_ref, o_ref):
    agg = aggp_ref[0] + aggp_ref[1]
    deg = degp_ref[0, :, 0:1] + degp_ref[1, :, 0:1]
    rdeg = 1.0 / jnp.maximum(deg, 1.0)
    h = jnp.dot(x_ref[...] + agg * rdeg, w_ref[...],
                preferred_element_type=jnp.float32) + b_ref[...]
    o_ref[...] = jnp.maximum(h, 0.0)


def _layer_tc(x, aggp, degp, w, b):
    grid = (N // _ROWS_TC,)
    return pl.pallas_call(
        _layer_tc_body,
        grid=grid,
        in_specs=[
            pl.BlockSpec((_ROWS_TC, D), lambda i: (i, 0)),
            pl.BlockSpec((NC, _ROWS_TC, D), lambda i: (0, i, 0)),
            pl.BlockSpec((NC, _ROWS_TC, D), lambda i: (0, i, 0)),
            pl.BlockSpec((D, D), lambda i: (0, 0)),
            pl.BlockSpec((1, D), lambda i: (0, 0)),
        ],
        out_specs=pl.BlockSpec((_ROWS_TC, D), lambda i: (i, 0)),
        out_shape=jax.ShapeDtypeStruct((N, D), jnp.float32),
    )(x, aggp, degp, w, b)


_ROWS_FIN = 512


def _final_tc_body(d0_ref, d1_ref, fw_ref, fb_ref, lw_ref, lb_ref, o_ref):
    o_ref[0] = jnp.dot(d0_ref[...], fw_ref[...],
                       preferred_element_type=jnp.float32) + fb_ref[...]
    o_ref[1] = jnp.dot(d1_ref[...], lw_ref[...],
                       preferred_element_type=jnp.float32) + lb_ref[...]


def _final_tc(d0rows, d1rows, fw, fb, lw, lb):
    grid = (K // _ROWS_FIN,)
    return pl.pallas_call(
        _final_tc_body,
        grid=grid,
        in_specs=[
            pl.BlockSpec((_ROWS_FIN, D), lambda i: (i, 0)),
            pl.BlockSpec((_ROWS_FIN, D), lambda i: (i, 0)),
            pl.BlockSpec((D, D), lambda i: (0, 0)),
            pl.BlockSpec((1, D), lambda i: (0, 0)),
            pl.BlockSpec((D, D), lambda i: (0, 0)),
            pl.BlockSpec((1, D), lambda i: (0, 0)),
        ],
        out_specs=pl.BlockSpec((2, _ROWS_FIN, D), lambda i: (0, i, 0)),
        out_shape=jax.ShapeDtypeStruct((2, K, D), jnp.float32),
    )(d0rows, d1rows, fw, fb, lw, lb)


_edge_extras = _make_edge_kernel(True)
_edge_plain = _make_edge_kernel(False)


def kernel(x0, x1, edge_index1, adj1_0, adj1_1, gnn1_W, gnn1_b, gnn2_W,
           gnn2_b, lin_W, lin_b, fcin_W, fcin_b):
    src = edge_index1[0]
    dst = edge_index1[1]
    flat0 = adj1_0[:, 0] * A + adj1_0[:, 1]
    flat1 = adj1_1[:, 0] * A + adj1_1[:, 1]
    zD = jnp.zeros((EC, D), jnp.float32)
    oD = jnp.ones((EC, D), jnp.float32)

    agg1, deg, d0rows = _edge_extras(x1, src, dst, x0, flat0, zD, oD)
    y1 = _layer_tc(x1, agg1, deg, gnn1_W, gnn1_b.reshape(1, D))
    (agg2,) = _edge_plain(y1, src, dst, zD)
    y2 = _layer_tc(y1, agg2, deg, gnn2_W, gnn2_b.reshape(1, D))
    (d1rows,) = _gather_kernel(y2, flat1)
    return _final_tc(d0rows, d1rows, fcin_W, fcin_b.reshape(1, D),
                     lin_W, lin_b.reshape(1, D))


# speedup vs baseline: 6.6536x; 1.1906x over previous
import functools

import jax
import jax.numpy as jnp
from jax import lax
from jax.experimental import pallas as pl
from jax.experimental.pallas import tpu as pltpu
from jax.experimental.pallas import tpu_sc as plsc

B = 100
A = 100
N = B * A
D = 128
E = 320000
K = 8192

NC = 2
NS = 16
NW = NC * NS
EC = 80
CHUNKS_PER_TILE = E // (NW * EC)
NP = 10240
ROWS_PER_TILE = NP // NS
GC = 64
KCH = K // (NW * GC)

_mesh = plsc.VectorSubcoreMesh(core_axis_name="c", subcore_axis_name="s")


def _edge_body(with_extras, *refs):
    if with_extras:
        (x_hbm, src_hbm, dst_hbm, x0_hbm, flat0_hbm, zD_hbm, oD_hbm,
         agg_out, deg_out, d0_out,
         agg_sh, rows0, rows1, sidx0, sidx1, didx0, didx1,
         sg0, sg1, ss0, ss1, sw0, sw1) = refs
    else:
        (x_hbm, src_hbm, dst_hbm, zD_hbm,
         agg_out,
         agg_sh, rows0, rows1, sidx0, sidx1, didx0, didx1,
         sg0, sg1, ss0, ss1, sw0, sw1) = refs

    c = lax.axis_index("c")
    s = lax.axis_index("s")
    wid = s * NC + c
    rows = (rows0, rows1)
    sidx = (sidx0, sidx1)
    didx = (didx0, didx1)
    sg = (sg0, sg1)
    ss = (ss0, ss1)
    sw = (sw0, sw1)

    def ebase(t):
        return (wid * CHUNKS_PER_TILE + t) * EC

    def zero_stripes():
        pltpu.sync_copy(zD_hbm, rows0)
        for k in range(ROWS_PER_TILE // EC):
            pltpu.sync_copy(
                rows0, agg_sh.at[pl.ds(s * ROWS_PER_TILE + k * EC, EC)])

    def copy_stripes_out(out_hbm):
        nck = ROWS_PER_TILE // EC
        for k in range(nck):
            b = k % 2
            r = s * ROWS_PER_TILE + k * EC
            if k >= 2:
                pltpu.make_async_copy(
                    rows[b], out_hbm.at[c, pl.ds(r - 2 * EC, EC)], sw[b]).wait()
            pltpu.sync_copy(agg_sh.at[pl.ds(r, EC)], rows[b])
            pltpu.async_copy(rows[b], out_hbm.at[c, pl.ds(r, EC)], sw[b])
        for k in range(nck - 2, nck):
            b = k % 2
            r = s * ROWS_PER_TILE + k * EC
            pltpu.make_async_copy(
                rows[b], out_hbm.at[c, pl.ds(r, EC)], sw[b]).wait()


    def issue_didx(t, b):
        pltpu.sync_copy(dst_hbm.at[pl.ds(ebase(t), EC)], didx[b])

    def start_gather(t, b):
        pltpu.sync_copy(src_hbm.at[pl.ds(ebase(t), EC)], sidx[b])
        pltpu.async_copy(x_hbm.at[sidx[b]], rows[b], sg[b])

    def wait_gather(b):
        pltpu.make_async_copy(x_hbm.at[sidx[b]], rows[b], sg[b]).wait()

    def agg_pass():
        issue_didx(0, 0)
        start_gather(0, 0)

        def pair(jj, carry):
            t0 = 2 * jj
            issue_didx(t0 + 1, 1)
            start_gather(t0 + 1, 1)
            wait_gather(0)
            d_s0 = pltpu.async_copy(rows0, agg_sh.at[didx0], ss0, add=True)
            wait_gather(1)
            d_s1 = pltpu.async_copy(rows1, agg_sh.at[didx1], ss1, add=True)
            d_s0.wait()
            issue_didx(t0 + 2, 0)
            start_gather(t0 + 2, 0)
            d_s1.wait()
            return carry

        npairs = (CHUNKS_PER_TILE - 1) // 2
        lax.fori_loop(0, npairs, pair, 0)

        wait_gather(0)
        pltpu.async_copy(rows0, agg_sh.at[didx0], ss0, add=True).wait()

    def ones_pass():
        def pair(jj, carry):
            t0 = 2 * jj
            issue_didx(t0, 0)
            d_s0 = pltpu.async_copy(rows0, agg_sh.at[didx0], ss0, add=True)
            issue_didx(t0 + 1, 1)
            d_s1 = pltpu.async_copy(rows0, agg_sh.at[didx1], ss1, add=True)
            d_s0.wait()
            d_s1.wait()
            return carry

        npairs = (CHUNKS_PER_TILE - 1) // 2
        lax.fori_loop(0, npairs, pair, 0)

        t = CHUNKS_PER_TILE - 1
        issue_didx(t, 0)
        pltpu.async_copy(rows0, agg_sh.at[didx0], ss0, add=True).wait()

    def k_gather(tbl_hbm, flat_hbm, out_hbm):
        for k in range(KCH):
            b = k % 2
            off = (wid * KCH + k) * GC
            if k >= 2:
                pltpu.make_async_copy(
                    rows[b].at[pl.ds(0, GC)],
                    out_hbm.at[pl.ds((wid * KCH + k - 2) * GC, GC)],
                    sw[b]).wait()
            pltpu.sync_copy(flat_hbm.at[pl.ds(off, GC)], sidx[b].at[pl.ds(0, GC)])
            pltpu.async_copy(
                tbl_hbm.at[sidx[b].at[pl.ds(0, GC)]],
                rows[b].at[pl.ds(0, GC)], sg[b]).wait()
            pltpu.async_copy(
                rows[b].at[pl.ds(0, GC)], out_hbm.at[pl.ds(off, GC)], sw[b])
        for k in range(KCH - 2, KCH):
            b = k % 2
            pltpu.make_async_copy(
                rows[b].at[pl.ds(0, GC)],
                out_hbm.at[pl.ds((wid * KCH + k) * GC, GC)], sw[b]).wait()


    zero_stripes()

    if with_extras:
        pltpu.sync_copy(oD_hbm, rows0)
        plsc.subcore_barrier()
        ones_pass()
        plsc.subcore_barrier()
        copy_stripes_out(deg_out)
        zero_stripes()
        k_gather(x0_hbm, flat0_hbm, d0_out)

    plsc.subcore_barrier()

    agg_pass()

    plsc.subcore_barrier()

    copy_stripes_out(agg_out)


def _make_edge_kernel(with_extras):
    out_type = [jax.ShapeDtypeStruct((NC, NP, D), jnp.float32)]
    scratch = [
        pltpu.MemorySpace.VMEM_SHARED((NP, D), jnp.float32),
        pltpu.VMEM((EC, D), jnp.float32),
        pltpu.VMEM((EC, D), jnp.float32),
        pltpu.VMEM((EC,), jnp.int32),
        pltpu.VMEM((EC,), jnp.int32),
        pltpu.VMEM((EC,), jnp.int32),
        pltpu.VMEM((EC,), jnp.int32),
        pltpu.SemaphoreType.DMA,
        pltpu.SemaphoreType.DMA,
        pltpu.SemaphoreType.DMA,
        pltpu.SemaphoreType.DMA,
        pltpu.SemaphoreType.DMA,
        pltpu.SemaphoreType.DMA,
    ]
    if with_extras:
        out_type += [jax.ShapeDtypeStruct((NC, NP, D), jnp.float32),
                     jax.ShapeDtypeStruct((K, D), jnp.float32)]
    return pl.kernel(
        functools.partial(_edge_body, with_extras),
        out_type=out_type,
        mesh=_mesh,
        scratch_types=scratch,
    )


def _gather_body(y_hbm, flat_hbm, out, sidx, rows, sem, sw0, sw1):
    c = lax.axis_index("c")
    s = lax.axis_index("s")
    wid = s * NC + c
    sw = (sw0, sw1)
    for k in range(KCH):
        b = k % 2
        off = (wid * KCH + k) * GC
        if k >= 2:
            pltpu.make_async_copy(
                rows.at[b], out.at[pl.ds((wid * KCH + k - 2) * GC, GC)],
                sw[b]).wait()
        pltpu.sync_copy(flat_hbm.at[pl.ds(off, GC)], sidx)
        pltpu.async_copy(y_hbm.at[sidx], rows.at[b], sem).wait()
        pltpu.async_copy(rows.at[b], out.at[pl.ds(off, GC)], sw[b])
    for k in range(KCH - 2, KCH):
        b = k % 2
        pltpu.make_async_copy(
            rows.at[b], out.at[pl.ds((wid * KCH + k) * GC, GC)], sw[b]).wait()


_gather_kernel = pl.kernel(
    _gather_body,
    out_type=[jax.ShapeDtypeStruct((K, D), jnp.float32)],
    mesh=_mesh,
    scratch_types=[
        pltpu.VMEM((GC,), jnp.int32),
        pltpu.VMEM((2, GC, D), jnp.float32),
        pltpu.SemaphoreType.DMA,
        pltpu.SemaphoreType.DMA,
        pltpu.SemaphoreType.DMA,
    ],
)

_ROWS_TC = 400


def _layer_tc_body(x_ref, aggp_ref, degp_ref, w_ref, b_ref, o_ref):
    agg = aggp_ref[0] + aggp_ref[1]
    deg = degp_ref[0, :, 0:1] + degp_ref[1, :, 0:1]
    rdeg = 1.0 / jnp.maximum(deg, 1.0)
    h = jnp.dot(x_ref[...] + agg * rdeg, w_ref[...],
                preferred_element_type=jnp.float32) + b_ref[...]
    o_ref[...] = jnp.maximum(h, 0.0)


def _layer_tc(x, aggp, degp, w, b):
    grid = (N // _ROWS_TC,)
    return pl.pallas_call(
        _layer_tc_body,
        grid=grid,
        in_specs=[
            pl.BlockSpec((_ROWS_TC, D), lambda i: (i, 0)),
            pl.BlockSpec((NC, _ROWS_TC, D), lambda i: (0, i, 0)),
            pl.BlockSpec((NC, _ROWS_TC, D), lambda i: (0, i, 0)),
            pl.BlockSpec((D, D), lambda i: (0, 0)),
            pl.BlockSpec((1, D), lambda i: (0, 0)),
        ],
        out_specs=pl.BlockSpec((_ROWS_TC, D), lambda i: (i, 0)),
        out_shape=jax.ShapeDtypeStruct((N, D), jnp.float32),
    )(x, aggp, degp, w, b)


_ROWS_FIN = 512


def _final_tc_body(d0_ref, d1_ref, fw_ref, fb_ref, lw_ref, lb_ref, o_ref):
    o_ref[0] = jnp.dot(d0_ref[...], fw_ref[...],
                       preferred_element_type=jnp.float32) + fb_ref[...]
    o_ref[1] = jnp.dot(d1_ref[...], lw_ref[...],
                       preferred_element_type=jnp.float32) + lb_ref[...]


def _final_tc(d0rows, d1rows, fw, fb, lw, lb):
    grid = (K // _ROWS_FIN,)
    return pl.pallas_call(
        _final_tc_body,
        grid=grid,
        in_specs=[
            pl.BlockSpec((_ROWS_FIN, D), lambda i: (i, 0)),
            pl.BlockSpec((_ROWS_FIN, D), lambda i: (i, 0)),
            pl.BlockSpec((D, D), lambda i: (0, 0)),
            pl.BlockSpec((1, D), lambda i: (0, 0)),
            pl.BlockSpec((D, D), lambda i: (0, 0)),
            pl.BlockSpec((1, D), lambda i: (0, 0)),
        ],
        out_specs=pl.BlockSpec((2, _ROWS_FIN, D), lambda i: (0, i, 0)),
        out_shape=jax.ShapeDtypeStruct((2, K, D), jnp.float32),
    )(d0rows, d1rows, fw, fb, lw, lb)


_edge_extras = _make_edge_kernel(True)
_edge_plain = _make_edge_kernel(False)


def kernel(x0, x1, edge_index1, adj1_0, adj1_1, gnn1_W, gnn1_b, gnn2_W,
           gnn2_b, lin_W, lin_b, fcin_W, fcin_b):
    src = edge_index1[0]
    dst = edge_index1[1]
    flat0 = adj1_0[:, 0] * A + adj1_0[:, 1]
    flat1 = adj1_1[:, 0] * A + adj1_1[:, 1]
    zD = jnp.zeros((EC, D), jnp.float32)
    oD = jnp.ones((EC, D), jnp.float32)

    agg1, deg, d0rows = _edge_extras(x1, src, dst, x0, flat0, zD, oD)
    y1 = _layer_tc(x1, agg1, deg, gnn1_W, gnn1_b.reshape(1, D))
    (agg2,) = _edge_plain(y1, src, dst, zD)
    y2 = _layer_tc(y1, agg2, deg, gnn2_W, gnn2_b.reshape(1, D))
    (d1rows,) = _gather_kernel(y2, flat1)
    return _final_tc(d0rows, d1rows, fcin_W, fcin_b.reshape(1, D),
                     lin_W, lin_b.reshape(1, D))
